# baseline (speedup 1.0000x reference)
import functools

import jax
import jax.numpy as jnp
from jax import lax
from jax.experimental import pallas as pl
from jax.experimental.pallas import tpu as pltpu
from jax.experimental.pallas import tpu_sc as plsc

@functools.lru_cache(maxsize=None)
def _make_gather(V, D, B):
    info = plsc.get_sparse_core_info()
    nc, ns = info.num_cores, info.num_subcores
    nw = nc * ns
    assert B % (8 * nw) == 0, (V, D, B)
    b_per_w = B // nw
    mesh = plsc.VectorSubcoreMesh(core_axis_name="c", subcore_axis_name="s")

    ch = 128
    while b_per_w % ch:
        ch //= 2
    nch = b_per_w // ch

    @functools.partial(
        pl.kernel,
        mesh=mesh,
        out_type=jax.ShapeDtypeStruct((B, D), jnp.float32),
        scratch_types=[
            pltpu.VMEM((b_per_w,), jnp.int32),
            pltpu.VMEM((b_per_w, D), jnp.float32),
        ]
        + [pltpu.SemaphoreType.DMA] * (2 * nch),
    )
    def gather_kernel(table_hbm, idx_hbm, out_hbm, idx_v, rows_v, *sems):
        gsems, wsems = sems[:nch], sems[nch:]
        wid = lax.axis_index("s") * nc + lax.axis_index("c")
        base = wid * b_per_w
        pltpu.sync_copy(idx_hbm.at[pl.ds(base, b_per_w)], idx_v)
        gathers = [
            pltpu.async_copy(
                table_hbm.at[idx_v.at[pl.ds(k * ch, ch)]],
                rows_v.at[pl.ds(k * ch, ch)],
                gsems[k],
            )
            for k in range(nch)
        ]
        writes = []
        for k in range(nch):
            gathers[k].wait()
            writes.append(
                pltpu.async_copy(
                    rows_v.at[pl.ds(k * ch, ch)],
                    out_hbm.at[pl.ds(base + k * ch, ch)],
                    wsems[k],
                )
            )
        for w in writes:
            w.wait()

    return gather_kernel


def kernel(x, n):
    V, D = x.shape
    (B,) = n.shape
    return _make_gather(V, D, B)(x, n.astype(jnp.int32))


# speedup vs baseline: 1.0139x; 1.0139x over previous
import functools

import jax
import jax.numpy as jnp
from jax import lax
from jax.experimental import pallas as pl
from jax.experimental.pallas import tpu as pltpu
from jax.experimental.pallas import tpu_sc as plsc

@functools.lru_cache(maxsize=None)
def _make_gather(V, D, B):
    info = plsc.get_sparse_core_info()
    nc, ns = info.num_cores, info.num_subcores
    nw = nc * ns
    assert B % (8 * nw) == 0, (V, D, B)
    b_per_w = B // nw
    mesh = plsc.VectorSubcoreMesh(core_axis_name="c", subcore_axis_name="s")

    nch = 2
    assert b_per_w % nch == 0
    ch = b_per_w // nch

    @functools.partial(
        pl.kernel,
        mesh=mesh,
        out_type=jax.ShapeDtypeStruct((B, D), jnp.float32),
        scratch_types=[
            pltpu.VMEM((b_per_w,), jnp.int32),
            pltpu.VMEM((b_per_w, D), jnp.float32),
        ]
        + [pltpu.SemaphoreType.DMA] * (2 * nch),
    )
    def gather_kernel(table_hbm, idx_hbm, out_hbm, idx_v, rows_v, *sems):
        gsems, wsems = sems[:nch], sems[nch:]
        wid = lax.axis_index("s") * nc + lax.axis_index("c")
        base = wid * b_per_w
        pltpu.sync_copy(idx_hbm.at[pl.ds(base, b_per_w)], idx_v)
        gathers = [
            pltpu.async_copy(
                table_hbm.at[idx_v.at[pl.ds(k * ch, ch)]],
                rows_v.at[pl.ds(k * ch, ch)],
                gsems[k],
            )
            for k in range(nch)
        ]
        writes = []
        for k in range(nch):
            gathers[k].wait()
            writes.append(
                pltpu.async_copy(
                    rows_v.at[pl.ds(k * ch, ch)],
                    out_hbm.at[pl.ds(base + k * ch, ch)],
                    wsems[k],
                )
            )
        for w in writes:
            w.wait()

    return gather_kernel


def kernel(x, n):
    V, D = x.shape
    (B,) = n.shape
    return _make_gather(V, D, B)(x, n.astype(jnp.int32))


# speedup vs baseline: 1.0270x; 1.0129x over previous
import functools

import jax
import jax.numpy as jnp
from jax import lax
from jax.experimental import pallas as pl
from jax.experimental.pallas import tpu as pltpu
from jax.experimental.pallas import tpu_sc as plsc

@functools.lru_cache(maxsize=None)
def _make_gather(V, D, B):
    info = plsc.get_sparse_core_info()
    nc, ns = info.num_cores, info.num_subcores
    nw = nc * ns
    assert B % (8 * nw) == 0, (V, D, B)
    b_per_w = B // nw
    mesh = plsc.VectorSubcoreMesh(core_axis_name="c", subcore_axis_name="s")

    @functools.partial(
        pl.kernel,
        mesh=mesh,
        out_type=jax.ShapeDtypeStruct((B, D), jnp.float32),
        scratch_types=[
            pltpu.VMEM((b_per_w,), jnp.int32),
            pltpu.VMEM((b_per_w, D), jnp.float32),
            pltpu.SemaphoreType.DMA,
        ],
    )
    def gather_kernel(table_hbm, idx_hbm, out_hbm, idx_v, rows_v, sem):
        wid = lax.axis_index("s") * nc + lax.axis_index("c")
        base = wid * b_per_w
        pltpu.sync_copy(idx_hbm.at[pl.ds(base, b_per_w)], idx_v)
        pltpu.async_copy(table_hbm.at[idx_v], rows_v, sem).wait()
        pltpu.sync_copy(rows_v, out_hbm.at[pl.ds(base, b_per_w)])

    return gather_kernel


def kernel(x, n):
    V, D = x.shape
    (B,) = n.shape
    return _make_gather(V, D, B)(x, n.astype(jnp.int32))
